# baseline (speedup 1.0000x reference)
import functools

import jax
import jax.numpy as jnp
from jax import lax
from jax.experimental import pallas as pl
from jax.experimental.pallas import tpu as pltpu
from jax.experimental.pallas import tpu_sc as plsc

N = 10000
E = 320000
F = 128
HID = 64

NC = 2
NS = 16
NW = NC * NS
CW = 64
CH = 160
EPW = CH * CW
NR = 10
CPR = CH // NR
NPAD = 10240
SLICE = NPAD // NS
TRASH = N

_mesh = plsc.VectorSubcoreMesh(
    core_axis_name="c", subcore_axis_name="s", num_cores=NC, num_subcores=NS)


def _mask_self_loops(srcv, dstv, out_ref, keep_ref):
    def row(i, _):
        def sub(j, _):
            sv = srcv[i, pl.ds(j * 16, 16)]
            dv = dstv[i, pl.ds(j * 16, 16)]
            kv = keep_ref[i, pl.ds(j * 16, 16)]
            out_ref[i, pl.ds(j * 16, 16)] = jnp.where(
                sv == dv, jnp.full((16,), TRASH, jnp.int32), kv)
            return 0
        return lax.fori_loop(0, CW // 16, sub, 0)
    lax.fori_loop(0, CPR, row, 0)


@functools.partial(
    pl.kernel,
    out_type=jax.ShapeDtypeStruct((NC * NPAD,), jnp.float32),
    mesh=_mesh,
    scratch_types=[
        pltpu.VMEM((NR, CPR, CW), jnp.int32),
        pltpu.VMEM((NR, CPR, CW), jnp.int32),
        pltpu.VMEM((CW,), jnp.float32),
        pltpu.VMEM_SHARED((NPAD,), jnp.float32),
    ],
)
def _sc_degree(src_hbm, dst_hbm, zero1_hbm, deg_out, srcv, dstv, onesv, deg_sh):
    c = lax.axis_index("c")
    s = lax.axis_index("s")
    w = s * NC + c
    pltpu.sync_copy(src_hbm.at[w], srcv)
    pltpu.sync_copy(dst_hbm.at[w], dstv)
    for j in range(CW // 16):
        onesv[pl.ds(j * 16, 16)] = jnp.ones((16,), jnp.float32)

    def mrow(k, _):
        r, i = lax.div(k, CPR), lax.rem(k, CPR)

        def sub(j, _):
            sv = srcv[r, i, pl.ds(j * 16, 16)]
            dv = dstv[r, i, pl.ds(j * 16, 16)]
            srcv[r, i, pl.ds(j * 16, 16)] = jnp.where(
                sv == dv, jnp.full((16,), TRASH, jnp.int32), sv)
            return 0
        return lax.fori_loop(0, CW // 16, sub, 0)
    lax.fori_loop(0, NR * CPR, mrow, 0)
    pltpu.sync_copy(zero1_hbm, deg_sh.at[pl.ds(s * SLICE, SLICE)])
    plsc.subcore_barrier()

    def chunk(k, _):
        r, i = lax.div(k, CPR), lax.rem(k, CPR)
        pltpu.sync_copy(onesv, deg_sh.at[srcv.at[r, i]], add=True)
        return 0
    lax.fori_loop(0, NR * CPR, chunk, 0)
    plsc.subcore_barrier()
    pltpu.sync_copy(deg_sh.at[pl.ds(s * SLICE, SLICE)],
                    deg_out.at[pl.ds(c * NPAD + s * SLICE, SLICE)])


@functools.partial(
    pl.kernel,
    out_type=jax.ShapeDtypeStruct((NC, NPAD, F), jnp.float32),
    mesh=_mesh,
    scratch_types=[
        pltpu.VMEM((CPR, CW), jnp.int32),
        pltpu.VMEM((CPR, CW), jnp.int32),
        pltpu.VMEM((4, CW, F), jnp.float32),
        pltpu.VMEM_SHARED((NPAD, F), jnp.float32),
        pltpu.SemaphoreType.DMA,
    ],
)
def _sc_scatter(src_hbm, dst_hbm, yw_hbm, zero2_hbm, s_out, srcv, dstv, rows,
                s_sh, gsem):
    c = lax.axis_index("c")
    s = lax.axis_index("s")
    w = s * NC + c
    pltpu.sync_copy(zero2_hbm, s_sh.at[pl.ds(s * SLICE, SLICE)])
    plsc.subcore_barrier()

    def rnd(r, _):
        pltpu.sync_copy(src_hbm.at[w, r], srcv)
        pltpu.sync_copy(dst_hbm.at[w, r], dstv)
        _mask_self_loops(srcv, dstv, dstv, dstv)
        pltpu.async_copy(yw_hbm.at[srcv.at[0]], rows.at[0], gsem)
        pltpu.async_copy(yw_hbm.at[srcv.at[1]], rows.at[1], gsem)
        pltpu.async_copy(yw_hbm.at[srcv.at[2]], rows.at[2], gsem)

        def chunk(i, _):
            b = lax.rem(i, 4)
            pltpu.make_async_copy(yw_hbm.at[srcv.at[i]], rows.at[b],
                                  gsem).wait()

            @pl.when(i + 3 < CPR)
            def _prefetch():
                pltpu.async_copy(yw_hbm.at[srcv.at[i + 3]],
                                 rows.at[lax.rem(i + 3, 4)], gsem)
            pltpu.sync_copy(rows.at[b], s_sh.at[dstv.at[i]], add=True)
            return 0
        lax.fori_loop(0, CPR, chunk, 0)
        return 0
    lax.fori_loop(0, NR, rnd, 0)
    plsc.subcore_barrier()
    pltpu.sync_copy(s_sh.at[pl.ds(s * SLICE, SLICE)],
                    s_out.at[c, pl.ds(s * SLICE, SLICE)])


_RB = 1000


def _tc_pre_body(x_ref, degt_ref, wcat_ref, yw_ref):
    degcol = degt_ref[...]
    dinv = jnp.where(degcol > 0, lax.rsqrt(degcol), 0.0)
    yw_ref[...] = (x_ref[...] * dinv) @ wcat_ref[...]


def _tc_post_body(x_ref, degt_ref, s_ref, w0_ref, bias_ref, linw_ref,
                  linb_ref, out_ref):
    degcol = degt_ref[...]
    dinv = jnp.where(degcol > 0, lax.rsqrt(degcol), 0.0)
    ssum = s_ref[0] + s_ref[1]
    a = x_ref[...] @ w0_ref[...] + bias_ref[...] - dinv * ssum
    z = jax.nn.sigmoid(a[:, :HID])
    ht = jnp.tanh(a[:, HID:])
    out_ref[...] = ((1.0 - z) * ht) @ linw_ref[...] + linb_ref[...]


def kernel(x, edge_index, W_x, W_h, b_x, b_h, lin_W, lin_b):
    out_dim = lin_W.shape[1]
    wcat = jnp.concatenate([W_x[0, 1], W_x[2, 1]], axis=1)
    w0cat = jnp.concatenate([W_x[0, 0], W_x[2, 0]], axis=1)
    bias = jnp.concatenate([b_x[0] + b_h[0], b_x[2] + b_h[2]])[None, :]
    linb2 = lin_b[None, :]

    epad = NW * EPW - E
    ei_pad = jnp.pad(edge_index, ((0, 0), (0, epad)), constant_values=TRASH)
    src_r = ei_pad[0].reshape(NW, NR, CPR, CW)
    dst_r = ei_pad[1].reshape(NW, NR, CPR, CW)
    zero1 = jnp.zeros((SLICE,), jnp.float32)
    zero2 = jnp.zeros((SLICE, F), jnp.float32)

    deg2 = _sc_degree(src_r, dst_r, zero1)
    degt = (deg2[:NPAD] + deg2[NPAD:])[:, None]

    grid = N // _RB
    yw = pl.pallas_call(
        _tc_pre_body,
        grid=(grid,),
        in_specs=[
            pl.BlockSpec((_RB, F), lambda i: (i, 0)),
            pl.BlockSpec((_RB, 1), lambda i: (i, 0)),
            pl.BlockSpec((F, F), lambda i: (0, 0)),
        ],
        out_specs=pl.BlockSpec((_RB, F), lambda i: (i, 0)),
        out_shape=jax.ShapeDtypeStruct((NPAD, F), jnp.float32),
    )(x, degt, wcat)

    s2 = _sc_scatter(src_r, dst_r, yw, zero2)

    out = pl.pallas_call(
        _tc_post_body,
        grid=(grid,),
        in_specs=[
            pl.BlockSpec((_RB, F), lambda i: (i, 0)),
            pl.BlockSpec((_RB, 1), lambda i: (i, 0)),
            pl.BlockSpec((NC, _RB, F), lambda i: (0, i, 0)),
            pl.BlockSpec((F, F), lambda i: (0, 0)),
            pl.BlockSpec((1, F), lambda i: (0, 0)),
            pl.BlockSpec((HID, out_dim), lambda i: (0, 0)),
            pl.BlockSpec((1, out_dim), lambda i: (0, 0)),
        ],
        out_specs=pl.BlockSpec((_RB, out_dim), lambda i: (i, 0)),
        out_shape=jax.ShapeDtypeStruct((N, out_dim), jnp.float32),
    )(x, degt, s2, w0cat, bias, lin_W, linb2)
    return out


# speedup vs baseline: 3.1097x; 3.1097x over previous
import functools

import jax
import jax.numpy as jnp
from jax import lax
from jax.experimental import pallas as pl
from jax.experimental.pallas import tpu as pltpu
from jax.experimental.pallas import tpu_sc as plsc

N = 10000
E = 320000
F = 128
HID = 64

NC = 2
NS = 16
NW = NC * NS
EPW = E // NW
CH = 125
CW = 80
NR = 5
CPR = CH // NR
NPAD = 10240
SLICE = NPAD // NS
TRASH = N

_mesh = plsc.VectorSubcoreMesh(
    core_axis_name="c", subcore_axis_name="s", num_cores=NC, num_subcores=NS)


@functools.partial(
    pl.kernel,
    out_type=[jax.ShapeDtypeStruct((NC, NPAD), jnp.float32),
              jax.ShapeDtypeStruct((NW, NR, CPR, CW), jnp.int32)],
    mesh=_mesh,
    scratch_types=[
        pltpu.VMEM((NR, CPR, CW), jnp.int32),
        pltpu.VMEM((NR, CPR, CW), jnp.int32),
        pltpu.VMEM((CW,), jnp.float32),
        pltpu.VMEM_SHARED((NPAD,), jnp.float32),
    ],
)
def _sc_degree(src_hbm, dst_hbm, zero1_hbm, deg_out, dstm_out, srcv, dstv,
               onesv, deg_sh):
    c = lax.axis_index("c")
    s = lax.axis_index("s")
    w = s * NC + c
    pltpu.sync_copy(src_hbm.at[w], srcv)
    pltpu.sync_copy(dst_hbm.at[w], dstv)
    for j in range(CW // 16):
        onesv[pl.ds(j * 16, 16)] = jnp.ones((16,), jnp.float32)

    def mrow(k, _):
        r, i = lax.div(k, CPR), lax.rem(k, CPR)

        def sub(j, _):
            sv = srcv[r, i, pl.ds(j * 16, 16)]
            dv = dstv[r, i, pl.ds(j * 16, 16)]
            eq = sv == dv
            tr = jnp.full((16,), TRASH, jnp.int32)
            srcv[r, i, pl.ds(j * 16, 16)] = jnp.where(eq, tr, sv)
            dstv[r, i, pl.ds(j * 16, 16)] = jnp.where(eq, tr, dv)
            return 0
        return lax.fori_loop(0, CW // 16, sub, 0)
    lax.fori_loop(0, NR * CPR, mrow, 0)
    pltpu.sync_copy(dstv, dstm_out.at[w])
    pltpu.sync_copy(zero1_hbm, deg_sh.at[pl.ds(s * SLICE, SLICE)])
    plsc.subcore_barrier()

    def chunk(k, _):
        r, i = lax.div(k, CPR), lax.rem(k, CPR)
        pltpu.sync_copy(onesv, deg_sh.at[srcv.at[r, i]], add=True)
        return 0
    lax.fori_loop(0, NR * CPR, chunk, 0)
    plsc.subcore_barrier()
    pltpu.sync_copy(deg_sh.at[pl.ds(s * SLICE, SLICE)],
                    deg_out.at[c, pl.ds(s * SLICE, SLICE)])


@functools.partial(
    pl.kernel,
    out_type=jax.ShapeDtypeStruct((NC, NPAD, F), jnp.float32),
    mesh=_mesh,
    scratch_types=[
        pltpu.VMEM((CPR, CW), jnp.int32),
        pltpu.VMEM((CPR, CW), jnp.int32),
        pltpu.VMEM((3, CW, F), jnp.float32),
        pltpu.VMEM_SHARED((NPAD, F), jnp.float32),
        pltpu.SemaphoreType.DMA,
    ],
)
def _sc_scatter(src_hbm, dst_hbm, yw_hbm, zero2_hbm, s_out, srcv, dstv, rows,
                s_sh, gsem):
    c = lax.axis_index("c")
    s = lax.axis_index("s")
    w = s * NC + c
    pltpu.sync_copy(zero2_hbm, s_sh.at[pl.ds(s * SLICE, SLICE)])
    plsc.subcore_barrier()

    def rnd(r, _):
        pltpu.sync_copy(src_hbm.at[w, r], srcv)
        pltpu.sync_copy(dst_hbm.at[w, r], dstv)
        pltpu.async_copy(yw_hbm.at[srcv.at[0]], rows.at[0], gsem)
        pltpu.async_copy(yw_hbm.at[srcv.at[1]], rows.at[1], gsem)

        def chunk(i, _):
            b = lax.rem(i, 3)
            pltpu.make_async_copy(yw_hbm.at[srcv.at[i]], rows.at[b],
                                  gsem).wait()

            @pl.when(i + 2 < CPR)
            def _prefetch():
                pltpu.async_copy(yw_hbm.at[srcv.at[i + 2]],
                                 rows.at[lax.rem(i + 2, 3)], gsem)
            pltpu.sync_copy(rows.at[b], s_sh.at[dstv.at[i]], add=True)
            return 0
        lax.fori_loop(0, CPR, chunk, 0)
        return 0
    lax.fori_loop(0, NR, rnd, 0)
    plsc.subcore_barrier()
    pltpu.sync_copy(s_sh.at[pl.ds(s * SLICE, SLICE)],
                    s_out.at[c, pl.ds(s * SLICE, SLICE)])


_RB = 1000


def _tc_pre_body(x_ref, degt_ref, wcat_ref, yw_ref):
    degcol = degt_ref[...]
    dinv = jnp.where(degcol > 0, lax.rsqrt(degcol), 0.0)
    yw_ref[...] = (x_ref[...] * dinv) @ wcat_ref[...]


def _tc_post_body(x_ref, degt_ref, s_ref, w0_ref, bias_ref, linw_ref,
                  linb_ref, out_ref):
    degcol = degt_ref[...]
    dinv = jnp.where(degcol > 0, lax.rsqrt(degcol), 0.0)
    ssum = s_ref[0] + s_ref[1]
    a = x_ref[...] @ w0_ref[...] + bias_ref[...] - dinv * ssum
    z = jax.nn.sigmoid(a[:, :HID])
    ht = jnp.tanh(a[:, HID:])
    out_ref[...] = ((1.0 - z) * ht) @ linw_ref[...] + linb_ref[...]


def kernel(x, edge_index, W_x, W_h, b_x, b_h, lin_W, lin_b):
    out_dim = lin_W.shape[1]
    wcat = jnp.concatenate([W_x[0, 1], W_x[2, 1]], axis=1)
    w0cat = jnp.concatenate([W_x[0, 0], W_x[2, 0]], axis=1)
    bias = jnp.concatenate([b_x[0] + b_h[0], b_x[2] + b_h[2]])[None, :]
    linb2 = lin_b[None, :]

    src_r = edge_index[0].reshape(NW, NR, CPR, CW)
    dst_r = edge_index[1].reshape(NW, NR, CPR, CW)
    zero1 = jnp.zeros((SLICE,), jnp.float32)
    zero2 = jnp.zeros((SLICE, F), jnp.float32)

    deg2, dstm = _sc_degree(src_r, dst_r, zero1)
    degt = (deg2[0] + deg2[1])[:, None]

    grid = N // _RB
    yw = pl.pallas_call(
        _tc_pre_body,
        grid=(grid,),
        in_specs=[
            pl.BlockSpec((_RB, F), lambda i: (i, 0)),
            pl.BlockSpec((_RB, 1), lambda i: (i, 0)),
            pl.BlockSpec((F, F), lambda i: (0, 0)),
        ],
        out_specs=pl.BlockSpec((_RB, F), lambda i: (i, 0)),
        out_shape=jax.ShapeDtypeStruct((N, F), jnp.float32),
    )(x, degt, wcat)

    s2 = _sc_scatter(src_r, dstm, yw, zero2)

    out = pl.pallas_call(
        _tc_post_body,
        grid=(grid,),
        in_specs=[
            pl.BlockSpec((_RB, F), lambda i: (i, 0)),
            pl.BlockSpec((_RB, 1), lambda i: (i, 0)),
            pl.BlockSpec((NC, _RB, F), lambda i: (0, i, 0)),
            pl.BlockSpec((F, F), lambda i: (0, 0)),
            pl.BlockSpec((1, F), lambda i: (0, 0)),
            pl.BlockSpec((HID, out_dim), lambda i: (0, 0)),
            pl.BlockSpec((1, out_dim), lambda i: (0, 0)),
        ],
        out_specs=pl.BlockSpec((_RB, out_dim), lambda i: (i, 0)),
        out_shape=jax.ShapeDtypeStruct((N, out_dim), jnp.float32),
    )(x, degt, s2, w0cat, bias, lin_W, linb2)
    return out


# speedup vs baseline: 3.1614x; 1.0166x over previous
import functools

import jax
import jax.numpy as jnp
from jax import lax
from jax.experimental import pallas as pl
from jax.experimental.pallas import tpu as pltpu
from jax.experimental.pallas import tpu_sc as plsc

N = 10000
E = 320000
F = 128
HID = 64

NC = 2
NS = 16
NW = NC * NS
EPW = E // NW
CH = 125
CW = 80
NR = 5
CPR = CH // NR
NPAD = 10240
SLICE = NPAD // NS
TRASH = N

_mesh = plsc.VectorSubcoreMesh(
    core_axis_name="c", subcore_axis_name="s", num_cores=NC, num_subcores=NS)


@functools.partial(
    pl.kernel,
    out_type=[jax.ShapeDtypeStruct((NC, NPAD), jnp.float32),
              jax.ShapeDtypeStruct((NW, NR, CPR, CW), jnp.int32)],
    mesh=_mesh,
    scratch_types=[
        pltpu.VMEM((NR, CPR, CW), jnp.int32),
        pltpu.VMEM((NR, CPR, CW), jnp.int32),
        pltpu.VMEM((CW,), jnp.float32),
        pltpu.VMEM_SHARED((NPAD,), jnp.float32),
    ],
)
def _sc_degree(src_hbm, dst_hbm, zero1_hbm, deg_out, dstm_out, srcv, dstv,
               onesv, deg_sh):
    c = lax.axis_index("c")
    s = lax.axis_index("s")
    w = s * NC + c
    pltpu.sync_copy(src_hbm.at[w], srcv)
    pltpu.sync_copy(dst_hbm.at[w], dstv)
    for j in range(CW // 16):
        onesv[pl.ds(j * 16, 16)] = jnp.ones((16,), jnp.float32)

    def mrow(k, _):
        r, i = lax.div(k, CPR), lax.rem(k, CPR)

        def sub(j, _):
            sv = srcv[r, i, pl.ds(j * 16, 16)]
            dv = dstv[r, i, pl.ds(j * 16, 16)]
            eq = sv == dv
            tr = jnp.full((16,), TRASH, jnp.int32)
            srcv[r, i, pl.ds(j * 16, 16)] = jnp.where(eq, tr, sv)
            dstv[r, i, pl.ds(j * 16, 16)] = jnp.where(eq, tr, dv)
            return 0
        return lax.fori_loop(0, CW // 16, sub, 0)
    lax.fori_loop(0, NR * CPR, mrow, 0)
    pltpu.sync_copy(dstv, dstm_out.at[w])
    pltpu.sync_copy(zero1_hbm, deg_sh.at[pl.ds(s * SLICE, SLICE)])
    plsc.subcore_barrier()

    def chunk(k, _):
        r, i = lax.div(k, CPR), lax.rem(k, CPR)
        pltpu.sync_copy(onesv, deg_sh.at[srcv.at[r, i]], add=True)
        return 0
    lax.fori_loop(0, NR * CPR, chunk, 0)
    plsc.subcore_barrier()
    pltpu.sync_copy(deg_sh.at[pl.ds(s * SLICE, SLICE)],
                    deg_out.at[c, pl.ds(s * SLICE, SLICE)])


@functools.partial(
    pl.kernel,
    out_type=jax.ShapeDtypeStruct((NC, NPAD, F), jnp.float32),
    mesh=_mesh,
    scratch_types=[
        pltpu.VMEM((CPR, CW), jnp.int32),
        pltpu.VMEM((CPR, CW), jnp.int32),
        pltpu.VMEM((3, CW, F), jnp.float32),
        pltpu.VMEM_SHARED((NPAD, F), jnp.float32),
        pltpu.SemaphoreType.DMA,
    ],
)
def _sc_scatter(src_hbm, dst_hbm, yw_hbm, zero2_hbm, s_out, srcv, dstv, rows,
                s_sh, gsem):
    c = lax.axis_index("c")
    s = lax.axis_index("s")
    w = s * NC + c
    pltpu.sync_copy(zero2_hbm, s_sh.at[pl.ds(s * SLICE, SLICE)])
    plsc.subcore_barrier()

    def rnd(r, _):
        pltpu.sync_copy(src_hbm.at[w, r], srcv)
        pltpu.sync_copy(dst_hbm.at[w, r], dstv)
        pltpu.async_copy(yw_hbm.at[srcv.at[0]], rows.at[0], gsem)
        pltpu.async_copy(yw_hbm.at[srcv.at[1]], rows.at[1], gsem)

        def chunk(i, _):
            b = lax.rem(i, 3)

            @pl.when(i + 2 < CPR)
            def _prefetch():
                pltpu.async_copy(yw_hbm.at[srcv.at[i + 2]],
                                 rows.at[lax.rem(i + 2, 3)], gsem)
            pltpu.make_async_copy(yw_hbm.at[srcv.at[i]], rows.at[b],
                                  gsem).wait()
            pltpu.sync_copy(rows.at[b], s_sh.at[dstv.at[i]], add=True)
            return 0
        lax.fori_loop(0, CPR, chunk, 0)
        return 0
    lax.fori_loop(0, NR, rnd, 0)
    plsc.subcore_barrier()
    pltpu.sync_copy(s_sh.at[pl.ds(s * SLICE, SLICE)],
                    s_out.at[c, pl.ds(s * SLICE, SLICE)])


_RB = 1000


def _tc_pre_body(x_ref, degt_ref, wcat_ref, yw_ref):
    degcol = degt_ref[...]
    dinv = jnp.where(degcol > 0, lax.rsqrt(degcol), 0.0)
    yw_ref[...] = (x_ref[...] * dinv) @ wcat_ref[...]


def _tc_post_body(x_ref, degt_ref, s_ref, w0_ref, bias_ref, linw_ref,
                  linb_ref, out_ref):
    degcol = degt_ref[...]
    dinv = jnp.where(degcol > 0, lax.rsqrt(degcol), 0.0)
    ssum = s_ref[0] + s_ref[1]
    a = x_ref[...] @ w0_ref[...] + bias_ref[...] - dinv * ssum
    z = jax.nn.sigmoid(a[:, :HID])
    ht = jnp.tanh(a[:, HID:])
    out_ref[...] = ((1.0 - z) * ht) @ linw_ref[...] + linb_ref[...]


def kernel(x, edge_index, W_x, W_h, b_x, b_h, lin_W, lin_b):
    out_dim = lin_W.shape[1]
    wcat = jnp.concatenate([W_x[0, 1], W_x[2, 1]], axis=1)
    w0cat = jnp.concatenate([W_x[0, 0], W_x[2, 0]], axis=1)
    bias = jnp.concatenate([b_x[0] + b_h[0], b_x[2] + b_h[2]])[None, :]
    linb2 = lin_b[None, :]

    src_r = edge_index[0].reshape(NW, NR, CPR, CW)
    dst_r = edge_index[1].reshape(NW, NR, CPR, CW)
    zero1 = jnp.zeros((SLICE,), jnp.float32)
    zero2 = jnp.zeros((SLICE, F), jnp.float32)

    deg2, dstm = _sc_degree(src_r, dst_r, zero1)
    degt = (deg2[0] + deg2[1])[:, None]

    grid = N // _RB
    yw = pl.pallas_call(
        _tc_pre_body,
        grid=(grid,),
        in_specs=[
            pl.BlockSpec((_RB, F), lambda i: (i, 0)),
            pl.BlockSpec((_RB, 1), lambda i: (i, 0)),
            pl.BlockSpec((F, F), lambda i: (0, 0)),
        ],
        out_specs=pl.BlockSpec((_RB, F), lambda i: (i, 0)),
        out_shape=jax.ShapeDtypeStruct((N, F), jnp.float32),
    )(x, degt, wcat)

    s2 = _sc_scatter(src_r, dstm, yw, zero2)

    out = pl.pallas_call(
        _tc_post_body,
        grid=(grid,),
        in_specs=[
            pl.BlockSpec((_RB, F), lambda i: (i, 0)),
            pl.BlockSpec((_RB, 1), lambda i: (i, 0)),
            pl.BlockSpec((NC, _RB, F), lambda i: (0, i, 0)),
            pl.BlockSpec((F, F), lambda i: (0, 0)),
            pl.BlockSpec((1, F), lambda i: (0, 0)),
            pl.BlockSpec((HID, out_dim), lambda i: (0, 0)),
            pl.BlockSpec((1, out_dim), lambda i: (0, 0)),
        ],
        out_specs=pl.BlockSpec((_RB, out_dim), lambda i: (i, 0)),
        out_shape=jax.ShapeDtypeStruct((N, out_dim), jnp.float32),
    )(x, degt, s2, w0cat, bias, lin_W, linb2)
    return out
